# baseline (speedup 1.0000x reference)
import jax
import jax.numpy as jnp
from jax import lax
from jax.experimental import pallas as pl
from jax.experimental.pallas import tpu as pltpu
from jax.experimental.pallas import tpu_sc as plsc

_N = 10000
_E = 320000
_D = 128
_NC, _NS, _L = 2, 16, 16
_NW = _NC * _NS
_EPW = _E // _NW
_CH = 80
_NCHUNK = _EPW // _CH
_NP = _N
_RPT = _NP // _NS
_NIB = 4
_NRB = 3
_PACK = 12
_MAIN = 120


def _make_segsum(with_deg):
    out_type = [jax.ShapeDtypeStruct((_NC, _NP, _D), jnp.float32)]
    scratch = [
        [pltpu.VMEM((_CH,), jnp.int32) for _ in range(_NIB)],
        [pltpu.VMEM((_CH,), jnp.int32) for _ in range(_NIB)],
        [pltpu.VMEM((_CH, _D), jnp.float32) for _ in range(_NRB)],
        pltpu.VMEM_SHARED((_NP, _D), jnp.float32),
        [pltpu.SemaphoreType.DMA for _ in range(_NIB)],
        [pltpu.SemaphoreType.DMA for _ in range(_NRB)],
        [pltpu.SemaphoreType.DMA for _ in range(_NRB)],
    ]
    if with_deg:
        out_type.append(jax.ShapeDtypeStruct((_NW, _NP), jnp.float32))
        scratch.insert(3, pltpu.VMEM((_NP,), jnp.float32))

    mesh = plsc.VectorSubcoreMesh(
        core_axis_name="c", subcore_axis_name="s",
        num_cores=_NC, num_subcores=_NS)

    def body(*refs):
        if with_deg:
            (table, src_h, dst_h, parts_out, deg_out,
             srcb, dstb, rows, deg_v, acc_sh, isem, gsem, ssem) = refs
        else:
            (table, src_h, dst_h, parts_out,
             srcb, dstb, rows, acc_sh, isem, gsem, ssem) = refs
        cid = lax.axis_index("c")
        sid = lax.axis_index("s")
        wid = sid * _NC + cid
        z16 = jnp.zeros((_L,), jnp.float32)
        ones = jnp.full((_L,), 1.0, jnp.float32)

        def zz(t, carry):
            rows[0][t // (_D // _L), pl.ds((t % (_D // _L)) * _L, _L)] = z16
            return carry
        lax.fori_loop(0, _CH * (_D // _L), zz, 0)
        for j in range(_RPT // _CH):
            pltpu.sync_copy(rows[0], acc_sh.at[pl.ds(sid * _RPT + j * _CH, _CH)])
        if _RPT % _CH:
            pltpu.sync_copy(
                rows[0].at[pl.ds(0, _RPT % _CH)],
                acc_sh.at[pl.ds(sid * _RPT + (_RPT // _CH) * _CH, _RPT % _CH)])
        if with_deg:
            def zd(t, carry):
                deg_v[pl.ds(t * _L, _L)] = z16
                return carry
            lax.fori_loop(0, _NP // _L, zd, 0)
        plsc.subcore_barrier()

        def fire_idx(g, bi):
            base = wid * _EPW + g * _CH
            pltpu.async_copy(src_h.at[pl.ds(base, _CH)], srcb[bi], isem[bi])
            pltpu.async_copy(dst_h.at[pl.ds(base, _CH)], dstb[bi], isem[bi])

        def wait_idx(bi):
            pltpu.make_async_copy(src_h.at[pl.ds(0, _CH)], srcb[bi], isem[bi]).wait()
            pltpu.make_async_copy(src_h.at[pl.ds(0, _CH)], dstb[bi], isem[bi]).wait()

        def fire_gather(bi, br):
            pltpu.async_copy(table.at[srcb[bi]], rows[br], gsem[br])

        def wait_gather(br, bi):
            pltpu.make_async_copy(table.at[srcb[bi]], rows[br], gsem[br]).wait()

        def fire_scat(br, bi):
            pltpu.async_copy(rows[br], acc_sh.at[dstb[bi]], ssem[br], add=True)

        def wait_scat(br):
            pltpu.make_async_copy(rows[br], acc_sh.at[dstb[0]], ssem[br]).wait()

        def hist(bi):
            if with_deg:
                for j in range(_CH // _L):
                    idx = dstb[bi][pl.ds(j * _L, _L)]
                    plsc.addupdate_scatter(deg_v, [idx], ones)

        def pstep(g, k, do_ssem=True, do_idx=True, do_gather=True):
            if do_ssem:
                wait_scat((k + 2) % _NRB)
            if do_idx:
                fire_idx(g + 3, (k + 3) % _NIB)
            if do_gather:
                wait_idx((k + 2) % _NIB)
                fire_gather((k + 2) % _NIB, (k + 2) % _NRB)
            wait_gather(k % _NRB, k % _NIB)
            fire_scat(k % _NRB, k % _NIB)
            hist(k % _NIB)

        for g in range(3):
            fire_idx(g, g)
        for g in range(2):
            wait_idx(g)
            fire_gather(g, g)
        for g in range(_PACK):
            pstep(g, g, do_ssem=(g >= 1))

        def pack_body(q, carry):
            for k in range(_PACK):
                pstep(q * _PACK + k, k)
            return carry
        lax.fori_loop(1, _MAIN // _PACK, pack_body, 0)

        for g in range(_MAIN, _NCHUNK):
            pstep(g, g % _PACK,
                  do_idx=(g + 3 < _NCHUNK), do_gather=(g + 2 < _NCHUNK))
        wait_scat((_NCHUNK - 1) % _NRB)

        plsc.subcore_barrier()
        pltpu.sync_copy(acc_sh.at[pl.ds(sid * _RPT, _RPT)],
                        parts_out.at[cid, pl.ds(sid * _RPT, _RPT)])
        if with_deg:
            pltpu.sync_copy(deg_v, deg_out.at[wid])

    return pl.kernel(body, out_type=tuple(out_type), mesh=mesh,
                     scratch_types=tuple(scratch),
                     compiler_params=pltpu.CompilerParams(
                         use_tc_tiling_on_sc=False,
                         needs_layout_passes=False))


_BN = 400


def _mm_bias_body(x_ref, w_ref, b_ref, o_ref):
    o_ref[...] = (jnp.dot(x_ref[...], w_ref[...],
                          preferred_element_type=jnp.float32)
                  + b_ref[...][None, :])


def _mm_bias(x, w, b):
    return pl.pallas_call(
        _mm_bias_body,
        grid=(_N // _BN,),
        in_specs=[
            pl.BlockSpec((_BN, _D), lambda i: (i, 0)),
            pl.BlockSpec((_D, _D), lambda i: (0, 0)),
            pl.BlockSpec((_D,), lambda i: (0,)),
        ],
        out_specs=pl.BlockSpec((_BN, _D), lambda i: (i, 0)),
        out_shape=jax.ShapeDtypeStruct((_N, _D), jnp.float32),
    )(x, w, b)


def _layer1_body(self_ref, p_ref, degp_ref, wn_ref, h1_ref, dinv_ref):
    agg = p_ref[0] + p_ref[1]
    deg = jnp.sum(degp_ref[...], axis=1)
    dinv = 1.0 / jnp.maximum(deg, 1.0)
    mean = agg * dinv[:, None]
    h = self_ref[...] + jnp.dot(mean, wn_ref[...],
                                preferred_element_type=jnp.float32)
    h1_ref[...] = jnp.maximum(h, 0.0)
    dinv_ref[...] = dinv[:, None]


def _layer2_body(self_ref, p_ref, dinv_ref, wn_ref,
                 wc1_ref, bc1_ref, wc2_ref, bc2_ref, out_ref):
    agg = p_ref[0] + p_ref[1]
    mean = agg * dinv_ref[...]
    h2 = self_ref[...] + jnp.dot(mean, wn_ref[...],
                                 preferred_element_type=jnp.float32)
    z = jnp.maximum(
        jnp.dot(h2, wc1_ref[...], preferred_element_type=jnp.float32)
        + bc1_ref[...][None, :], 0.0)
    out_ref[...] = (jnp.dot(z, wc2_ref[...], preferred_element_type=jnp.float32)
                    + bc2_ref[...][None, :])


def _layer1(self1, parts, degp, W_neigh):
    return pl.pallas_call(
        _layer1_body,
        grid=(_N // _BN,),
        in_specs=[
            pl.BlockSpec((_BN, _D), lambda i: (i, 0)),
            pl.BlockSpec((_NC, _BN, _D), lambda i: (0, i, 0)),
            pl.BlockSpec((_BN, _NW), lambda i: (i, 0)),
            pl.BlockSpec((_D, _D), lambda i: (0, 0)),
        ],
        out_specs=[
            pl.BlockSpec((_BN, _D), lambda i: (i, 0)),
            pl.BlockSpec((_BN, 1), lambda i: (i, 0)),
        ],
        out_shape=[
            jax.ShapeDtypeStruct((_N, _D), jnp.float32),
            jax.ShapeDtypeStruct((_N, 1), jnp.float32),
        ],
    )(self1, parts, degp, W_neigh)


def _layer2(self2, parts, dinv, W_neigh, Wc1, bc1, Wc2, bc2):
    return pl.pallas_call(
        _layer2_body,
        grid=(_N // _BN,),
        in_specs=[
            pl.BlockSpec((_BN, _D), lambda i: (i, 0)),
            pl.BlockSpec((_NC, _BN, _D), lambda i: (0, i, 0)),
            pl.BlockSpec((_BN, 1), lambda i: (i, 0)),
            pl.BlockSpec((_D, _D), lambda i: (0, 0)),
            pl.BlockSpec((_D, _D), lambda i: (0, 0)),
            pl.BlockSpec((_D,), lambda i: (0,)),
            pl.BlockSpec((_D, 2), lambda i: (0, 0)),
            pl.BlockSpec((2,), lambda i: (0,)),
        ],
        out_specs=pl.BlockSpec((_BN, 2), lambda i: (i, 0)),
        out_shape=jax.ShapeDtypeStruct((_N, 2), jnp.float32),
    )(self2, parts, dinv, W_neigh, Wc1, bc1, Wc2, bc2)


@jax.jit
def kernel(x, edge_index, W_self1, W_neigh1, b1, W_self2, W_neigh2, b2,
           Wc1, bc1, Wc2, bc2):
    src = edge_index[0]
    dst = edge_index[1]
    self1 = _mm_bias(x, W_self1, b1)
    parts1, degp = _make_segsum(True)(x, src, dst)
    h1, dinv = _layer1(self1, parts1, degp.T, W_neigh1)
    self2 = _mm_bias(h1, W_self2, b2)
    res2 = _make_segsum(False)(h1, src, dst)
    parts2 = res2[0] if isinstance(res2, (tuple, list)) else res2
    return _layer2(self2, parts2, dinv, W_neigh2, Wc1, bc1, Wc2, bc2)


# speedup vs baseline: 1.1140x; 1.1140x over previous
import jax
import jax.numpy as jnp
from jax import lax
from jax.experimental import pallas as pl
from jax.experimental.pallas import tpu as pltpu
from jax.experimental.pallas import tpu_sc as plsc

_N = 10000
_E = 320000
_D = 128
_NC, _NS, _L = 2, 16, 16
_NW = _NC * _NS
_EPW = _E // _NW
_CH = 80
_NCHUNK = _EPW // _CH
_NP = _N
_RPT = _NP // _NS
_NIB = 4
_NRB = 3
_PACK = 12
_MAIN = 120


def _make_segsum(with_deg):
    out_type = [jax.ShapeDtypeStruct((_NC, _NP, _D), jnp.float32)]
    scratch = [
        [pltpu.VMEM((_CH,), jnp.int32) for _ in range(_NIB)],
        [pltpu.VMEM((_CH,), jnp.int32) for _ in range(_NIB)],
        [pltpu.VMEM((_CH, _D), jnp.float32) for _ in range(_NRB)],
        pltpu.VMEM_SHARED((_NP, _D), jnp.float32),
        [pltpu.SemaphoreType.DMA for _ in range(_NIB)],
        [pltpu.SemaphoreType.DMA for _ in range(_NRB)],
        [pltpu.SemaphoreType.DMA for _ in range(_NRB)],
    ]
    if with_deg:
        out_type.append(jax.ShapeDtypeStruct((_NW, _NP), jnp.float32))
        scratch.insert(3, pltpu.VMEM((_NP,), jnp.float32))

    mesh = plsc.VectorSubcoreMesh(
        core_axis_name="c", subcore_axis_name="s",
        num_cores=_NC, num_subcores=_NS)

    def body(*refs):
        if with_deg:
            (table, ei_h, parts_out, deg_out,
             srcb, dstb, rows, deg_v, acc_sh, isem, gsem, ssem) = refs
        else:
            (table, ei_h, parts_out,
             srcb, dstb, rows, acc_sh, isem, gsem, ssem) = refs
        cid = lax.axis_index("c")
        sid = lax.axis_index("s")
        wid = sid * _NC + cid
        z16 = jnp.zeros((_L,), jnp.float32)
        ones = jnp.full((_L,), 1.0, jnp.float32)

        def zz(t, carry):
            rows[0][t // (_D // _L), pl.ds((t % (_D // _L)) * _L, _L)] = z16
            return carry
        lax.fori_loop(0, _CH * (_D // _L), zz, 0)
        for j in range(_RPT // _CH):
            pltpu.sync_copy(rows[0], acc_sh.at[pl.ds(sid * _RPT + j * _CH, _CH)])
        if _RPT % _CH:
            pltpu.sync_copy(
                rows[0].at[pl.ds(0, _RPT % _CH)],
                acc_sh.at[pl.ds(sid * _RPT + (_RPT // _CH) * _CH, _RPT % _CH)])
        if with_deg:
            def zd(t, carry):
                deg_v[pl.ds(t * _L, _L)] = z16
                return carry
            lax.fori_loop(0, _NP // _L, zd, 0)
        plsc.subcore_barrier()

        def fire_idx(g, bi):
            base = wid * _EPW + g * _CH
            pltpu.async_copy(ei_h.at[0, pl.ds(base, _CH)], srcb[bi], isem[bi])
            pltpu.async_copy(ei_h.at[1, pl.ds(base, _CH)], dstb[bi], isem[bi])

        def wait_idx(bi):
            pltpu.make_async_copy(ei_h.at[0, pl.ds(0, _CH)], srcb[bi], isem[bi]).wait()
            pltpu.make_async_copy(ei_h.at[0, pl.ds(0, _CH)], dstb[bi], isem[bi]).wait()

        def fire_gather(bi, br):
            pltpu.async_copy(table.at[srcb[bi]], rows[br], gsem[br])

        def wait_gather(br, bi):
            pltpu.make_async_copy(table.at[srcb[bi]], rows[br], gsem[br]).wait()

        def fire_scat(br, bi):
            pltpu.async_copy(rows[br], acc_sh.at[dstb[bi]], ssem[br], add=True)

        def wait_scat(br):
            pltpu.make_async_copy(rows[br], acc_sh.at[dstb[0]], ssem[br]).wait()

        def hist(bi):
            if with_deg:
                for j in range(_CH // _L):
                    idx = dstb[bi][pl.ds(j * _L, _L)]
                    plsc.addupdate_scatter(deg_v, [idx], ones)

        def pstep(g, k, do_ssem=True, do_idx=True, do_gather=True):
            if do_ssem:
                wait_scat((k + 2) % _NRB)
            if do_idx:
                fire_idx(g + 3, (k + 3) % _NIB)
            if do_gather:
                wait_idx((k + 2) % _NIB)
                fire_gather((k + 2) % _NIB, (k + 2) % _NRB)
            wait_gather(k % _NRB, k % _NIB)
            fire_scat(k % _NRB, k % _NIB)
            hist(k % _NIB)

        for g in range(3):
            fire_idx(g, g)
        for g in range(2):
            wait_idx(g)
            fire_gather(g, g)
        for g in range(_PACK):
            pstep(g, g, do_ssem=(g >= 1))

        def pack_body(q, carry):
            for k in range(_PACK):
                pstep(q * _PACK + k, k)
            return carry
        lax.fori_loop(1, _MAIN // _PACK, pack_body, 0)

        for g in range(_MAIN, _NCHUNK):
            pstep(g, g % _PACK,
                  do_idx=(g + 3 < _NCHUNK), do_gather=(g + 2 < _NCHUNK))
        wait_scat((_NCHUNK - 1) % _NRB)

        plsc.subcore_barrier()
        pltpu.sync_copy(acc_sh.at[pl.ds(sid * _RPT, _RPT)],
                        parts_out.at[cid, pl.ds(sid * _RPT, _RPT)])
        if with_deg:
            pltpu.sync_copy(deg_v, deg_out.at[wid])

    return pl.kernel(body, out_type=tuple(out_type), mesh=mesh,
                     scratch_types=tuple(scratch),
                     compiler_params=pltpu.CompilerParams(
                         use_tc_tiling_on_sc=False,
                         needs_layout_passes=False))


_BN = 1000


def _layer1_body(x_ref, p_ref, degp_ref, ws_ref, wn_ref, b_ref,
                 h1_ref, dinv_ref):
    agg = p_ref[0] + p_ref[1]
    deg = jnp.sum(degp_ref[...], axis=1)
    dinv = 1.0 / jnp.maximum(deg, 1.0)
    mean = agg * dinv[:, None]
    h = (jnp.dot(x_ref[...], ws_ref[...], preferred_element_type=jnp.float32)
         + jnp.dot(mean, wn_ref[...], preferred_element_type=jnp.float32)
         + b_ref[...][None, :])
    h1_ref[...] = jnp.maximum(h, 0.0)
    dinv_ref[...] = dinv[:, None]


def _layer2_body(h1_ref, p_ref, dinv_ref, ws_ref, wn_ref, b_ref,
                 wc1_ref, bc1_ref, wc2_ref, bc2_ref, out_ref):
    agg = p_ref[0] + p_ref[1]
    mean = agg * dinv_ref[...]
    h2 = (jnp.dot(h1_ref[...], ws_ref[...], preferred_element_type=jnp.float32)
          + jnp.dot(mean, wn_ref[...], preferred_element_type=jnp.float32)
          + b_ref[...][None, :])
    z = jnp.maximum(
        jnp.dot(h2, wc1_ref[...], preferred_element_type=jnp.float32)
        + bc1_ref[...][None, :], 0.0)
    out_ref[...] = (jnp.dot(z, wc2_ref[...], preferred_element_type=jnp.float32)
                    + bc2_ref[...][None, :])


def _layer1(x, parts, degp, W_self, W_neigh, b):
    return pl.pallas_call(
        _layer1_body,
        grid=(_N // _BN,),
        in_specs=[
            pl.BlockSpec((_BN, _D), lambda i: (i, 0)),
            pl.BlockSpec((_NC, _BN, _D), lambda i: (0, i, 0)),
            pl.BlockSpec((_BN, _NW), lambda i: (i, 0)),
            pl.BlockSpec((_D, _D), lambda i: (0, 0)),
            pl.BlockSpec((_D, _D), lambda i: (0, 0)),
            pl.BlockSpec((_D,), lambda i: (0,)),
        ],
        out_specs=[
            pl.BlockSpec((_BN, _D), lambda i: (i, 0)),
            pl.BlockSpec((_BN, 1), lambda i: (i, 0)),
        ],
        out_shape=[
            jax.ShapeDtypeStruct((_N, _D), jnp.float32),
            jax.ShapeDtypeStruct((_N, 1), jnp.float32),
        ],
    )(x, parts, degp, W_self, W_neigh, b)


def _layer2(h1, parts, dinv, W_self, W_neigh, b, Wc1, bc1, Wc2, bc2):
    return pl.pallas_call(
        _layer2_body,
        grid=(_N // _BN,),
        in_specs=[
            pl.BlockSpec((_BN, _D), lambda i: (i, 0)),
            pl.BlockSpec((_NC, _BN, _D), lambda i: (0, i, 0)),
            pl.BlockSpec((_BN, 1), lambda i: (i, 0)),
            pl.BlockSpec((_D, _D), lambda i: (0, 0)),
            pl.BlockSpec((_D, _D), lambda i: (0, 0)),
            pl.BlockSpec((_D,), lambda i: (0,)),
            pl.BlockSpec((_D, _D), lambda i: (0, 0)),
            pl.BlockSpec((_D,), lambda i: (0,)),
            pl.BlockSpec((_D, 2), lambda i: (0, 0)),
            pl.BlockSpec((2,), lambda i: (0,)),
        ],
        out_specs=pl.BlockSpec((_BN, 2), lambda i: (i, 0)),
        out_shape=jax.ShapeDtypeStruct((_N, 2), jnp.float32),
    )(h1, parts, dinv, W_self, W_neigh, b, Wc1, bc1, Wc2, bc2)


@jax.jit
def kernel(x, edge_index, W_self1, W_neigh1, b1, W_self2, W_neigh2, b2,
           Wc1, bc1, Wc2, bc2):
    parts1, degp = _make_segsum(True)(x, edge_index)
    h1, dinv = _layer1(x, parts1, degp.T, W_self1, W_neigh1, b1)
    res2 = _make_segsum(False)(h1, edge_index)
    parts2 = res2[0] if isinstance(res2, (tuple, list)) else res2
    return _layer2(h1, parts2, dinv, W_self2, W_neigh2, b2, Wc1, bc1, Wc2, bc2)


# speedup vs baseline: 1.1177x; 1.0034x over previous
import jax
import jax.numpy as jnp
from jax import lax
from jax.experimental import pallas as pl
from jax.experimental.pallas import tpu as pltpu
from jax.experimental.pallas import tpu_sc as plsc

_N = 10000
_E = 320000
_D = 128
_NC, _NS, _L = 2, 16, 16
_NW = _NC * _NS
_EPW = _E // _NW
_CH = 80
_NCHUNK = _EPW // _CH
_NP = _N
_RPT = _NP // _NS
_NIB = 4
_NRB = 3
_PACK = 12
_MAIN = 120


def _make_segsum(with_deg):
    out_type = [jax.ShapeDtypeStruct((_NC, _NP, _D), jnp.float32)]
    scratch = [
        [pltpu.VMEM((2, _CH), jnp.int32) for _ in range(_NIB)],
        [pltpu.VMEM((_CH, _D), jnp.float32) for _ in range(_NRB)],
        pltpu.VMEM_SHARED((_NP, _D), jnp.float32),
        [pltpu.SemaphoreType.DMA for _ in range(_NIB)],
        [pltpu.SemaphoreType.DMA for _ in range(_NRB)],
        [pltpu.SemaphoreType.DMA for _ in range(_NRB)],
    ]
    if with_deg:
        out_type.append(jax.ShapeDtypeStruct((_NW, _NP), jnp.float32))
        scratch.insert(2, pltpu.VMEM((_NP,), jnp.float32))

    mesh = plsc.VectorSubcoreMesh(
        core_axis_name="c", subcore_axis_name="s",
        num_cores=_NC, num_subcores=_NS)

    def body(*refs):
        if with_deg:
            (table, ei_h, parts_out, deg_out,
             idxb, rows, deg_v, acc_sh, isem, gsem, ssem) = refs
        else:
            (table, ei_h, parts_out,
             idxb, rows, acc_sh, isem, gsem, ssem) = refs
        cid = lax.axis_index("c")
        sid = lax.axis_index("s")
        wid = sid * _NC + cid
        z16 = jnp.zeros((_L,), jnp.float32)
        ones = jnp.full((_L,), 1.0, jnp.float32)

        def zz(t, carry):
            rows[0][t // (_D // _L), pl.ds((t % (_D // _L)) * _L, _L)] = z16
            return carry
        lax.fori_loop(0, _CH * (_D // _L), zz, 0)
        for j in range(_RPT // _CH):
            pltpu.sync_copy(rows[0], acc_sh.at[pl.ds(sid * _RPT + j * _CH, _CH)])
        if _RPT % _CH:
            pltpu.sync_copy(
                rows[0].at[pl.ds(0, _RPT % _CH)],
                acc_sh.at[pl.ds(sid * _RPT + (_RPT // _CH) * _CH, _RPT % _CH)])
        if with_deg:
            def zd(t, carry):
                deg_v[pl.ds(t * _L, _L)] = z16
                return carry
            lax.fori_loop(0, _NP // _L, zd, 0)
        plsc.subcore_barrier()

        def fire_idx(g, bi):
            base = wid * _EPW + g * _CH
            pltpu.async_copy(ei_h.at[:, pl.ds(base, _CH)], idxb[bi], isem[bi])

        def wait_idx(bi):
            pltpu.make_async_copy(ei_h.at[:, pl.ds(0, _CH)], idxb[bi], isem[bi]).wait()

        def fire_gather(bi, br):
            pltpu.async_copy(table.at[idxb[bi].at[0]], rows[br], gsem[br])

        def wait_gather(br, bi):
            pltpu.make_async_copy(table.at[idxb[bi].at[0]], rows[br], gsem[br]).wait()

        def fire_scat(br, bi):
            pltpu.async_copy(rows[br], acc_sh.at[idxb[bi].at[1]], ssem[br], add=True)

        def wait_scat(br):
            pltpu.make_async_copy(rows[br], acc_sh.at[idxb[0].at[1]], ssem[br]).wait()

        def hist(bi):
            if with_deg:
                for j in range(_CH // _L):
                    idx = idxb[bi][1, pl.ds(j * _L, _L)]
                    plsc.addupdate_scatter(deg_v, [idx], ones)

        def pstep(g, k, do_ssem=True, do_idx=True, do_gather=True):
            if do_ssem:
                wait_scat((k + 2) % _NRB)
            if do_idx:
                fire_idx(g + 3, (k + 3) % _NIB)
            if do_gather:
                wait_idx((k + 2) % _NIB)
                fire_gather((k + 2) % _NIB, (k + 2) % _NRB)
            wait_gather(k % _NRB, k % _NIB)
            fire_scat(k % _NRB, k % _NIB)
            hist(k % _NIB)

        for g in range(3):
            fire_idx(g, g)
        for g in range(2):
            wait_idx(g)
            fire_gather(g, g)
        for g in range(_PACK):
            pstep(g, g, do_ssem=(g >= 1))

        def pack_body(q, carry):
            for k in range(_PACK):
                pstep(q * _PACK + k, k)
            return carry
        lax.fori_loop(1, _MAIN // _PACK, pack_body, 0)

        for g in range(_MAIN, _NCHUNK):
            pstep(g, g % _PACK,
                  do_idx=(g + 3 < _NCHUNK), do_gather=(g + 2 < _NCHUNK))
        wait_scat((_NCHUNK - 1) % _NRB)

        plsc.subcore_barrier()
        pltpu.sync_copy(acc_sh.at[pl.ds(sid * _RPT, _RPT)],
                        parts_out.at[cid, pl.ds(sid * _RPT, _RPT)])
        if with_deg:
            pltpu.sync_copy(deg_v, deg_out.at[wid])

    return pl.kernel(body, out_type=tuple(out_type), mesh=mesh,
                     scratch_types=tuple(scratch),
                     compiler_params=pltpu.CompilerParams(
                         use_tc_tiling_on_sc=False,
                         needs_layout_passes=False))


_BN = 1000


def _layer1_body(x_ref, p_ref, degp_ref, ws_ref, wn_ref, b_ref,
                 h1_ref, dinv_ref):
    agg = p_ref[0] + p_ref[1]
    deg = jnp.sum(degp_ref[...], axis=1)
    dinv = 1.0 / jnp.maximum(deg, 1.0)
    mean = agg * dinv[:, None]
    h = (jnp.dot(x_ref[...], ws_ref[...], preferred_element_type=jnp.float32)
         + jnp.dot(mean, wn_ref[...], preferred_element_type=jnp.float32)
         + b_ref[...][None, :])
    h1_ref[...] = jnp.maximum(h, 0.0)
    dinv_ref[...] = dinv[:, None]


def _layer2_body(h1_ref, p_ref, dinv_ref, ws_ref, wn_ref, b_ref,
                 wc1_ref, bc1_ref, wc2_ref, bc2_ref, out_ref):
    agg = p_ref[0] + p_ref[1]
    mean = agg * dinv_ref[...]
    h2 = (jnp.dot(h1_ref[...], ws_ref[...], preferred_element_type=jnp.float32)
          + jnp.dot(mean, wn_ref[...], preferred_element_type=jnp.float32)
          + b_ref[...][None, :])
    z = jnp.maximum(
        jnp.dot(h2, wc1_ref[...], preferred_element_type=jnp.float32)
        + bc1_ref[...][None, :], 0.0)
    out_ref[...] = (jnp.dot(z, wc2_ref[...], preferred_element_type=jnp.float32)
                    + bc2_ref[...][None, :])


def _layer1(x, parts, degp, W_self, W_neigh, b):
    return pl.pallas_call(
        _layer1_body,
        grid=(_N // _BN,),
        in_specs=[
            pl.BlockSpec((_BN, _D), lambda i: (i, 0)),
            pl.BlockSpec((_NC, _BN, _D), lambda i: (0, i, 0)),
            pl.BlockSpec((_BN, _NW), lambda i: (i, 0)),
            pl.BlockSpec((_D, _D), lambda i: (0, 0)),
            pl.BlockSpec((_D, _D), lambda i: (0, 0)),
            pl.BlockSpec((_D,), lambda i: (0,)),
        ],
        out_specs=[
            pl.BlockSpec((_BN, _D), lambda i: (i, 0)),
            pl.BlockSpec((_BN, 1), lambda i: (i, 0)),
        ],
        out_shape=[
            jax.ShapeDtypeStruct((_N, _D), jnp.float32),
            jax.ShapeDtypeStruct((_N, 1), jnp.float32),
        ],
    )(x, parts, degp, W_self, W_neigh, b)


def _layer2(h1, parts, dinv, W_self, W_neigh, b, Wc1, bc1, Wc2, bc2):
    return pl.pallas_call(
        _layer2_body,
        grid=(_N // _BN,),
        in_specs=[
            pl.BlockSpec((_BN, _D), lambda i: (i, 0)),
            pl.BlockSpec((_NC, _BN, _D), lambda i: (0, i, 0)),
            pl.BlockSpec((_BN, 1), lambda i: (i, 0)),
            pl.BlockSpec((_D, _D), lambda i: (0, 0)),
            pl.BlockSpec((_D, _D), lambda i: (0, 0)),
            pl.BlockSpec((_D,), lambda i: (0,)),
            pl.BlockSpec((_D, _D), lambda i: (0, 0)),
            pl.BlockSpec((_D,), lambda i: (0,)),
            pl.BlockSpec((_D, 2), lambda i: (0, 0)),
            pl.BlockSpec((2,), lambda i: (0,)),
        ],
        out_specs=pl.BlockSpec((_BN, 2), lambda i: (i, 0)),
        out_shape=jax.ShapeDtypeStruct((_N, 2), jnp.float32),
    )(h1, parts, dinv, W_self, W_neigh, b, Wc1, bc1, Wc2, bc2)


@jax.jit
def kernel(x, edge_index, W_self1, W_neigh1, b1, W_self2, W_neigh2, b2,
           Wc1, bc1, Wc2, bc2):
    parts1, degp = _make_segsum(True)(x, edge_index)
    h1, dinv = _layer1(x, parts1, degp.T, W_self1, W_neigh1, b1)
    res2 = _make_segsum(False)(h1, edge_index)
    parts2 = res2[0] if isinstance(res2, (tuple, list)) else res2
    return _layer2(h1, parts2, dinv, W_self2, W_neigh2, b2, Wc1, bc1, Wc2, bc2)


# speedup vs baseline: 1.1396x; 1.0195x over previous
import jax
import jax.numpy as jnp
from jax import lax
from jax.experimental import pallas as pl
from jax.experimental.pallas import tpu as pltpu
from jax.experimental.pallas import tpu_sc as plsc

_N = 10000
_E = 320000
_D = 128
_NC, _NS, _L = 2, 16, 16
_NW = _NC * _NS
_EPW = _E // _NW
_CH = 80
_NCHUNK = _EPW // _CH
_NP = _N
_RPT = _NP // _NS
_NIB = 4
_NRB = 3
_PACK = 12
_MAIN = 120


def _make_segsum(with_deg):
    out_type = [jax.ShapeDtypeStruct((_NC, _NP, _D), jnp.float32)]
    scratch = [
        [pltpu.VMEM((2, _CH), jnp.int32) for _ in range(_NIB)],
        [pltpu.VMEM((_CH, _D), jnp.float32) for _ in range(_NRB)],
        pltpu.VMEM_SHARED((_NP, _D), jnp.float32),
        [pltpu.SemaphoreType.DMA for _ in range(_NIB)],
        [pltpu.SemaphoreType.DMA for _ in range(_NRB)],
        [pltpu.SemaphoreType.DMA for _ in range(_NRB)],
    ]
    if with_deg:
        out_type.append(jax.ShapeDtypeStruct((_NW, _NP), jnp.float32))
        scratch.insert(2, pltpu.VMEM((_NP,), jnp.float32))

    mesh = plsc.VectorSubcoreMesh(
        core_axis_name="c", subcore_axis_name="s",
        num_cores=_NC, num_subcores=_NS)

    def body(*refs):
        if with_deg:
            (table, ei_h, parts_out, deg_out,
             idxb, rows, deg_v, acc_sh, isem, gsem, ssem) = refs
        else:
            (table, ei_h, parts_out,
             idxb, rows, acc_sh, isem, gsem, ssem) = refs
        cid = lax.axis_index("c")
        sid = lax.axis_index("s")
        wid = sid * _NC + cid
        z16 = jnp.zeros((_L,), jnp.float32)
        ones = jnp.full((_L,), 1.0, jnp.float32)

        def zz(t, carry):
            rows[0][t // (_D // _L), pl.ds((t % (_D // _L)) * _L, _L)] = z16
            return carry
        lax.fori_loop(0, _CH * (_D // _L), zz, 0)
        for j in range(_RPT // _CH):
            pltpu.sync_copy(rows[0], acc_sh.at[pl.ds(sid * _RPT + j * _CH, _CH)])
        if _RPT % _CH:
            pltpu.sync_copy(
                rows[0].at[pl.ds(0, _RPT % _CH)],
                acc_sh.at[pl.ds(sid * _RPT + (_RPT // _CH) * _CH, _RPT % _CH)])
        if with_deg:
            def zd(t, carry):
                deg_v[pl.ds(t * _L, _L)] = z16
                return carry
            lax.fori_loop(0, _NP // _L, zd, 0)
        plsc.subcore_barrier()

        def fire_idx(g, bi):
            base = wid * _EPW + g * _CH
            pltpu.async_copy(ei_h.at[:, pl.ds(base, _CH)], idxb[bi], isem[bi])

        def wait_idx(bi):
            pltpu.make_async_copy(ei_h.at[:, pl.ds(0, _CH)], idxb[bi], isem[bi]).wait()

        def fire_gather(bi, br):
            pltpu.async_copy(table.at[idxb[bi].at[0]], rows[br], gsem[br])

        def wait_gather(br, bi):
            pltpu.make_async_copy(table.at[idxb[bi].at[0]], rows[br], gsem[br]).wait()

        def fire_scat(br, bi):
            pltpu.async_copy(rows[br], acc_sh.at[idxb[bi].at[1]], ssem[br], add=True)

        def wait_scat(br):
            pltpu.make_async_copy(rows[br], acc_sh.at[idxb[0].at[1]], ssem[br]).wait()

        def hist(bi):
            if with_deg:
                for j in range(_CH // _L):
                    idx = idxb[bi][1, pl.ds(j * _L, _L)]
                    plsc.addupdate_scatter(deg_v, [idx], ones)

        def pstep(g, k, do_ssem=True, do_idx=True, do_gather=True):
            if do_ssem:
                wait_scat((k + 2) % _NRB)
            if do_idx:
                fire_idx(g + 3, (k + 3) % _NIB)
            if do_gather:
                wait_idx((k + 2) % _NIB)
                fire_gather((k + 2) % _NIB, (k + 2) % _NRB)
            wait_gather(k % _NRB, k % _NIB)
            fire_scat(k % _NRB, k % _NIB)
            hist(k % _NIB)

        for g in range(3):
            fire_idx(g, g)
        for g in range(2):
            wait_idx(g)
            fire_gather(g, g)
        for g in range(_PACK):
            pstep(g, g, do_ssem=(g >= 1))

        def pack_body(q, carry):
            for k in range(_PACK):
                pstep(q * _PACK + k, k)
            return carry
        lax.fori_loop(1, _MAIN // _PACK, pack_body, 0)

        for g in range(_MAIN, _NCHUNK):
            pstep(g, g % _PACK,
                  do_idx=(g + 3 < _NCHUNK), do_gather=(g + 2 < _NCHUNK))
        wait_scat((_NCHUNK - 1) % _NRB)

        plsc.subcore_barrier()
        pltpu.sync_copy(acc_sh.at[pl.ds(sid * _RPT, _RPT)],
                        parts_out.at[cid, pl.ds(sid * _RPT, _RPT)])
        if with_deg:
            pltpu.sync_copy(deg_v, deg_out.at[wid])

    return pl.kernel(body, out_type=tuple(out_type), mesh=mesh,
                     scratch_types=tuple(scratch),
                     compiler_params=pltpu.CompilerParams(
                         use_tc_tiling_on_sc=False,
                         needs_layout_passes=False))


_BN = 2000


def _layer1_body(x_ref, p_ref, degp_ref, ws_ref, wn_ref, b_ref,
                 h1_ref, dinv_ref):
    agg = p_ref[0] + p_ref[1]
    deg = jnp.sum(degp_ref[...], axis=1)
    dinv = 1.0 / jnp.maximum(deg, 1.0)
    mean = agg * dinv[:, None]
    h = (jnp.dot(x_ref[...], ws_ref[...], preferred_element_type=jnp.float32)
         + jnp.dot(mean, wn_ref[...], preferred_element_type=jnp.float32)
         + b_ref[...][None, :])
    h1_ref[...] = jnp.maximum(h, 0.0)
    dinv_ref[...] = dinv[:, None]


def _layer2_body(h1_ref, p_ref, dinv_ref, ws_ref, wn_ref, b_ref,
                 wc1_ref, bc1_ref, wc2_ref, bc2_ref, out_ref):
    agg = p_ref[0] + p_ref[1]
    mean = agg * dinv_ref[...]
    h2 = (jnp.dot(h1_ref[...], ws_ref[...], preferred_element_type=jnp.float32)
          + jnp.dot(mean, wn_ref[...], preferred_element_type=jnp.float32)
          + b_ref[...][None, :])
    z = jnp.maximum(
        jnp.dot(h2, wc1_ref[...], preferred_element_type=jnp.float32)
        + bc1_ref[...][None, :], 0.0)
    out_ref[...] = (jnp.dot(z, wc2_ref[...], preferred_element_type=jnp.float32)
                    + bc2_ref[...][None, :])


def _layer1(x, parts, degp, W_self, W_neigh, b):
    return pl.pallas_call(
        _layer1_body,
        grid=(_N // _BN,),
        in_specs=[
            pl.BlockSpec((_BN, _D), lambda i: (i, 0)),
            pl.BlockSpec((_NC, _BN, _D), lambda i: (0, i, 0)),
            pl.BlockSpec((_BN, _NW), lambda i: (i, 0)),
            pl.BlockSpec((_D, _D), lambda i: (0, 0)),
            pl.BlockSpec((_D, _D), lambda i: (0, 0)),
            pl.BlockSpec((_D,), lambda i: (0,)),
        ],
        out_specs=[
            pl.BlockSpec((_BN, _D), lambda i: (i, 0)),
            pl.BlockSpec((_BN, 1), lambda i: (i, 0)),
        ],
        out_shape=[
            jax.ShapeDtypeStruct((_N, _D), jnp.float32),
            jax.ShapeDtypeStruct((_N, 1), jnp.float32),
        ],
    )(x, parts, degp, W_self, W_neigh, b)


def _layer2(h1, parts, dinv, W_self, W_neigh, b, Wc1, bc1, Wc2, bc2):
    return pl.pallas_call(
        _layer2_body,
        grid=(_N // _BN,),
        in_specs=[
            pl.BlockSpec((_BN, _D), lambda i: (i, 0)),
            pl.BlockSpec((_NC, _BN, _D), lambda i: (0, i, 0)),
            pl.BlockSpec((_BN, 1), lambda i: (i, 0)),
            pl.BlockSpec((_D, _D), lambda i: (0, 0)),
            pl.BlockSpec((_D, _D), lambda i: (0, 0)),
            pl.BlockSpec((_D,), lambda i: (0,)),
            pl.BlockSpec((_D, _D), lambda i: (0, 0)),
            pl.BlockSpec((_D,), lambda i: (0,)),
            pl.BlockSpec((_D, 2), lambda i: (0, 0)),
            pl.BlockSpec((2,), lambda i: (0,)),
        ],
        out_specs=pl.BlockSpec((_BN, 2), lambda i: (i, 0)),
        out_shape=jax.ShapeDtypeStruct((_N, 2), jnp.float32),
    )(h1, parts, dinv, W_self, W_neigh, b, Wc1, bc1, Wc2, bc2)


@jax.jit
def kernel(x, edge_index, W_self1, W_neigh1, b1, W_self2, W_neigh2, b2,
           Wc1, bc1, Wc2, bc2):
    parts1, degp = _make_segsum(True)(x, edge_index)
    h1, dinv = _layer1(x, parts1, degp.T, W_self1, W_neigh1, b1)
    res2 = _make_segsum(False)(h1, edge_index)
    parts2 = res2[0] if isinstance(res2, (tuple, list)) else res2
    return _layer2(h1, parts2, dinv, W_self2, W_neigh2, b2, Wc1, bc1, Wc2, bc2)


# speedup vs baseline: 1.1455x; 1.0052x over previous
import jax
import jax.numpy as jnp
from jax import lax
from jax.experimental import pallas as pl
from jax.experimental.pallas import tpu as pltpu
from jax.experimental.pallas import tpu_sc as plsc

_N = 10000
_E = 320000
_D = 128
_NC, _NS, _L = 2, 16, 16
_NW = _NC * _NS
_EPW = _E // _NW
_CH = 80
_NCHUNK = _EPW // _CH
_NP = _N
_RPT = _NP // _NS
_NIB = 4
_NRB = 3
_PACK = 12
_MAIN = 120


def _make_segsum(with_deg):
    out_type = [jax.ShapeDtypeStruct((_NC, _NP, _D), jnp.float32)]
    scratch = [
        [pltpu.VMEM((2, _CH), jnp.int32) for _ in range(_NIB)],
        [pltpu.VMEM((_CH, _D), jnp.float32) for _ in range(_NRB)],
        pltpu.VMEM_SHARED((_NP, _D), jnp.float32),
        [pltpu.SemaphoreType.DMA for _ in range(_NIB)],
        [pltpu.SemaphoreType.DMA for _ in range(_NRB)],
        [pltpu.SemaphoreType.DMA for _ in range(_NRB)],
    ]
    if with_deg:
        out_type.append(jax.ShapeDtypeStruct((_NW, _NP), jnp.float32))
        scratch.insert(2, pltpu.VMEM((_NP,), jnp.float32))

    mesh = plsc.VectorSubcoreMesh(
        core_axis_name="c", subcore_axis_name="s",
        num_cores=_NC, num_subcores=_NS)

    def body(*refs):
        if with_deg:
            (table, ei_h, parts_out, deg_out,
             idxb, rows, deg_v, acc_sh, isem, gsem, ssem) = refs
        else:
            (table, ei_h, parts_out,
             idxb, rows, acc_sh, isem, gsem, ssem) = refs
        cid = lax.axis_index("c")
        sid = lax.axis_index("s")
        wid = sid * _NC + cid
        z16 = jnp.zeros((_L,), jnp.float32)
        ones = jnp.full((_L,), 1.0, jnp.float32)

        def zz(t, carry):
            rows[0][t // (_D // _L), pl.ds((t % (_D // _L)) * _L, _L)] = z16
            return carry
        lax.fori_loop(0, _CH * (_D // _L), zz, 0)
        for j in range(_RPT // _CH):
            pltpu.sync_copy(rows[0], acc_sh.at[pl.ds(sid * _RPT + j * _CH, _CH)])
        if _RPT % _CH:
            pltpu.sync_copy(
                rows[0].at[pl.ds(0, _RPT % _CH)],
                acc_sh.at[pl.ds(sid * _RPT + (_RPT // _CH) * _CH, _RPT % _CH)])
        if with_deg:
            def zd(t, carry):
                deg_v[pl.ds(t * _L, _L)] = z16
                return carry
            lax.fori_loop(0, _NP // _L, zd, 0)
        plsc.subcore_barrier()

        def fire_idx(g, bi):
            base = wid * _EPW + g * _CH
            pltpu.async_copy(ei_h.at[:, pl.ds(base, _CH)], idxb[bi], isem[bi])

        def wait_idx(bi):
            pltpu.make_async_copy(ei_h.at[:, pl.ds(0, _CH)], idxb[bi], isem[bi]).wait()

        def fire_gather(bi, br):
            pltpu.async_copy(table.at[idxb[bi].at[0]], rows[br], gsem[br])

        def wait_gather(br, bi):
            pltpu.make_async_copy(table.at[idxb[bi].at[0]], rows[br], gsem[br]).wait()

        def fire_scat(br, bi):
            pltpu.async_copy(rows[br], acc_sh.at[idxb[bi].at[1]], ssem[br], add=True)

        def wait_scat(br):
            pltpu.make_async_copy(rows[br], acc_sh.at[idxb[0].at[1]], ssem[br]).wait()

        def hist(bi):
            if with_deg:
                for j in range(_CH // _L):
                    idx = idxb[bi][1, pl.ds(j * _L, _L)]
                    plsc.addupdate_scatter(deg_v, [idx], ones)

        def pstep(g, k, do_ssem=True, do_idx=True, do_gather=True):
            if do_ssem:
                wait_scat((k + 2) % _NRB)
            if do_idx:
                fire_idx(g + 3, (k + 3) % _NIB)
            if do_gather:
                wait_idx((k + 2) % _NIB)
                fire_gather((k + 2) % _NIB, (k + 2) % _NRB)
            wait_gather(k % _NRB, k % _NIB)
            fire_scat(k % _NRB, k % _NIB)
            hist(k % _NIB)

        for g in range(3):
            fire_idx(g, g)
        for g in range(2):
            wait_idx(g)
            fire_gather(g, g)
        for g in range(_PACK):
            pstep(g, g, do_ssem=(g >= 1))

        def pack_body(q, carry):
            for k in range(_PACK):
                pstep(q * _PACK + k, k)
            return carry
        lax.fori_loop(1, _MAIN // _PACK, pack_body, 0)

        for g in range(_MAIN, _NCHUNK):
            pstep(g, g % _PACK,
                  do_idx=(g + 3 < _NCHUNK), do_gather=(g + 2 < _NCHUNK))
        wait_scat((_NCHUNK - 1) % _NRB)

        plsc.subcore_barrier()
        pltpu.sync_copy(acc_sh.at[pl.ds(sid * _RPT, _RPT)],
                        parts_out.at[cid, pl.ds(sid * _RPT, _RPT)])
        if with_deg:
            pltpu.sync_copy(deg_v, deg_out.at[wid])

    return pl.kernel(body, out_type=tuple(out_type), mesh=mesh,
                     scratch_types=tuple(scratch),
                     compiler_params=pltpu.CompilerParams(
                         use_tc_tiling_on_sc=False,
                         needs_layout_passes=False))


_BN = 5000


def _layer1_body(x_ref, p_ref, degp_ref, ws_ref, wn_ref, b_ref,
                 h1_ref, dinv_ref):
    agg = p_ref[0] + p_ref[1]
    deg = jnp.sum(degp_ref[...], axis=1)
    dinv = 1.0 / jnp.maximum(deg, 1.0)
    mean = agg * dinv[:, None]
    h = (jnp.dot(x_ref[...], ws_ref[...], preferred_element_type=jnp.float32)
         + jnp.dot(mean, wn_ref[...], preferred_element_type=jnp.float32)
         + b_ref[...][None, :])
    h1_ref[...] = jnp.maximum(h, 0.0)
    dinv_ref[...] = dinv[:, None]


def _layer2_body(h1_ref, p_ref, dinv_ref, ws_ref, wn_ref, b_ref,
                 wc1_ref, bc1_ref, wc2_ref, bc2_ref, out_ref):
    agg = p_ref[0] + p_ref[1]
    mean = agg * dinv_ref[...]
    h2 = (jnp.dot(h1_ref[...], ws_ref[...], preferred_element_type=jnp.float32)
          + jnp.dot(mean, wn_ref[...], preferred_element_type=jnp.float32)
          + b_ref[...][None, :])
    z = jnp.maximum(
        jnp.dot(h2, wc1_ref[...], preferred_element_type=jnp.float32)
        + bc1_ref[...][None, :], 0.0)
    out_ref[...] = (jnp.dot(z, wc2_ref[...], preferred_element_type=jnp.float32)
                    + bc2_ref[...][None, :])


def _layer1(x, parts, degp, W_self, W_neigh, b):
    return pl.pallas_call(
        _layer1_body,
        grid=(_N // _BN,),
        in_specs=[
            pl.BlockSpec((_BN, _D), lambda i: (i, 0)),
            pl.BlockSpec((_NC, _BN, _D), lambda i: (0, i, 0)),
            pl.BlockSpec((_BN, _NW), lambda i: (i, 0)),
            pl.BlockSpec((_D, _D), lambda i: (0, 0)),
            pl.BlockSpec((_D, _D), lambda i: (0, 0)),
            pl.BlockSpec((_D,), lambda i: (0,)),
        ],
        out_specs=[
            pl.BlockSpec((_BN, _D), lambda i: (i, 0)),
            pl.BlockSpec((_BN, 1), lambda i: (i, 0)),
        ],
        out_shape=[
            jax.ShapeDtypeStruct((_N, _D), jnp.float32),
            jax.ShapeDtypeStruct((_N, 1), jnp.float32),
        ],
    )(x, parts, degp, W_self, W_neigh, b)


def _layer2(h1, parts, dinv, W_self, W_neigh, b, Wc1, bc1, Wc2, bc2):
    return pl.pallas_call(
        _layer2_body,
        grid=(_N // _BN,),
        in_specs=[
            pl.BlockSpec((_BN, _D), lambda i: (i, 0)),
            pl.BlockSpec((_NC, _BN, _D), lambda i: (0, i, 0)),
            pl.BlockSpec((_BN, 1), lambda i: (i, 0)),
            pl.BlockSpec((_D, _D), lambda i: (0, 0)),
            pl.BlockSpec((_D, _D), lambda i: (0, 0)),
            pl.BlockSpec((_D,), lambda i: (0,)),
            pl.BlockSpec((_D, _D), lambda i: (0, 0)),
            pl.BlockSpec((_D,), lambda i: (0,)),
            pl.BlockSpec((_D, 2), lambda i: (0, 0)),
            pl.BlockSpec((2,), lambda i: (0,)),
        ],
        out_specs=pl.BlockSpec((_BN, 2), lambda i: (i, 0)),
        out_shape=jax.ShapeDtypeStruct((_N, 2), jnp.float32),
    )(h1, parts, dinv, W_self, W_neigh, b, Wc1, bc1, Wc2, bc2)


@jax.jit
def kernel(x, edge_index, W_self1, W_neigh1, b1, W_self2, W_neigh2, b2,
           Wc1, bc1, Wc2, bc2):
    parts1, degp = _make_segsum(True)(x, edge_index)
    h1, dinv = _layer1(x, parts1, degp.T, W_self1, W_neigh1, b1)
    res2 = _make_segsum(False)(h1, edge_index)
    parts2 = res2[0] if isinstance(res2, (tuple, list)) else res2
    return _layer2(h1, parts2, dinv, W_self2, W_neigh2, b2, Wc1, bc1, Wc2, bc2)
